# baseline (speedup 1.0000x reference)
import functools

import jax
import jax.numpy as jnp
from jax import lax
from jax.experimental import pallas as pl
from jax.experimental.pallas import tpu as pltpu
from jax.experimental.pallas import tpu_sc as plsc

B = 4096
CTX = 20
D = 64
VOCAB = 1_000_000
LEC = 100_000
NS = 5

NB = 16_384
BW = 64
PAD = NB * BW
L = 16


def _build_sc_call(num_cores: int, num_subcores: int):
  nw = num_cores * num_subcores
  b_per = B // nw
  samp = b_per * NS
  cb = NB // num_subcores
  st = cb // 2

  mesh = plsc.VectorSubcoreMesh(
      core_axis_name="c", subcore_axis_name="s",
      num_cores=num_cores, num_subcores=num_subcores)

  @functools.partial(
      pl.kernel,
      out_type=[
          jax.ShapeDtypeStruct((B, D), jnp.float32),
          jax.ShapeDtypeStruct((B, D), jnp.float32),
          jax.ShapeDtypeStruct((B * NS, D), jnp.float32),
      ],
      mesh=mesh,
      compiler_params=pltpu.CompilerParams(
          needs_layout_passes=False, use_tc_tiling_on_sc=False),
      scratch_types=[
          pltpu.VMEM((b_per * (CTX + 1),), jnp.int32),
          pltpu.VMEM((samp, D), jnp.float32),
          pltpu.VMEM((st, BW), jnp.float32),
          pltpu.VMEM((cb,), jnp.float32),
          pltpu.VMEM((NB,), jnp.float32),
          pltpu.VMEM_SHARED((NB,), jnp.float32),
          pltpu.VMEM((samp,), jnp.float32),
          pltpu.VMEM((samp,), jnp.int32),
          pltpu.VMEM((samp,), jnp.int32),
          pltpu.VMEM((b_per,), jnp.int32),
          pltpu.VMEM((b_per,), jnp.int32),
          pltpu.VMEM((b_per * CTX,), jnp.int32),
          pltpu.VMEM((b_per, D), jnp.float32),
          pltpu.VMEM((b_per, D), jnp.float32),
          pltpu.SemaphoreType.DMA,
      ],
  )
  def sc_call(inputs_hbm, target_hbm, lecture_hbm, wemb_hbm, cdf2d_hbm, u_hbm,
              dvec_hbm, tgt_hbm, nvec_hbm,
              inputs_v, rows_v, stage_v, cc_v, coarse_v, shared_coarse,
              u_v, bucket_v, nw_v, doc_v, tgt_iv, ctx_iv, lec_v, dvec_v, sem):
    c = lax.axis_index("c")
    s = lax.axis_index("s")
    wid = s * num_cores + c
    base_b = wid * b_per
    w = CTX + 1

    pltpu.sync_copy(inputs_hbm.at[pl.ds(base_b * w, b_per * w)], inputs_v)
    pltpu.sync_copy(target_hbm.at[pl.ds(base_b, b_per)], tgt_iv)
    pltpu.sync_copy(u_hbm.at[pl.ds(wid * samp, samp)], u_v)

    def doc_loop(g, _):
      rows = lax.iota(jnp.int32, L) + g * L
      doc_v[pl.ds(g * L, L)] = plsc.load_gather(inputs_v, [rows * w])
      return 0
    lax.fori_loop(0, b_per // L, doc_loop, 0)

    def ctx_loop(g, _):
      p = lax.iota(jnp.int32, L) + g * L
      r = p // CTX
      cc = p - r * CTX + 1
      ctx_iv[pl.ds(g * L, L)] = plsc.load_gather(inputs_v, [r * w + cc])
      return 0
    lax.fori_loop(0, (b_per * CTX) // L, ctx_loop, 0)

    for h in range(2):
      pltpu.sync_copy(cdf2d_hbm.at[pl.ds(s * cb + h * st, st)], stage_v)

      def ext_loop(g, _, h=h):
        rows = lax.iota(jnp.int32, L) + g * L
        cols = jnp.full((L,), BW - 1, jnp.int32)
        cc_v[pl.ds(h * st + g * L, L)] = plsc.load_gather(
            stage_v, [rows, cols])
        return 0
      lax.fori_loop(0, st // L, ext_loop, 0)

    pltpu.sync_copy(cc_v, shared_coarse.at[pl.ds(s * cb, cb)])
    plsc.subcore_barrier()
    pltpu.sync_copy(shared_coarse, coarse_v)

    def coarse_search(g, _):
      u = u_v[pl.ds(g * L, L)]
      lo = jnp.zeros((L,), jnp.int32)
      hi = jnp.full((L,), NB - 1, jnp.int32)
      for _ in range(14):
        mid = (lo + hi) >> 1
        vals = plsc.load_gather(coarse_v, [mid])
        pred = vals < u
        lo = jnp.where(pred, mid + 1, lo)
        hi = jnp.where(pred, hi, mid)
      bucket_v[pl.ds(g * L, L)] = lo
      return 0
    lax.fori_loop(0, samp // L, coarse_search, 0)

    win = [
        pltpu.make_async_copy(
            cdf2d_hbm.at[bucket_v.at[pl.ds(j * 128, 128)]],
            rows_v.at[pl.ds(j * 128, 128)], sem)
        for j in range(samp // 128)
    ]
    for cp in win:
      cp.start()
    for cp in win:
      cp.wait()

    def fine_search(g, _):
      u = u_v[pl.ds(g * L, L)]
      rows = lax.iota(jnp.int32, L) + g * L
      lo = jnp.zeros((L,), jnp.int32)
      hi = jnp.full((L,), BW - 1, jnp.int32)
      for _ in range(6):
        mid = (lo + hi) >> 1
        vals = plsc.load_gather(rows_v, [rows, mid])
        pred = vals < u
        lo = jnp.where(pred, mid + 1, lo)
        hi = jnp.where(pred, hi, mid)
      bucket = bucket_v[pl.ds(g * L, L)]
      word = jnp.minimum(bucket * BW + lo,
                         jnp.full((L,), VOCAB - 1, jnp.int32))
      nw_v[pl.ds(g * L, L)] = word
      return 0
    lax.fori_loop(0, samp // L, fine_search, 0)

    neg = [
        pltpu.make_async_copy(
            wemb_hbm.at[nw_v.at[pl.ds(j * 128, 128)]],
            rows_v.at[pl.ds(j * 128, 128)], sem)
        for j in range(samp // 128)
    ]
    for cp in neg:
      cp.start()
    for cp in neg:
      cp.wait()

    def neg_loop(r, _):
      for q in range(D // L):
        rows_v[r, pl.ds(q * L, L)] = -rows_v[r, pl.ds(q * L, L)]
      return 0
    lax.fori_loop(0, samp, neg_loop, 0)
    pltpu.sync_copy(rows_v, nvec_hbm.at[pl.ds(wid * samp, samp)])

    tgt_cp = pltpu.make_async_copy(
        wemb_hbm.at[tgt_iv], rows_v.at[pl.ds(0, b_per)], sem)
    tgt_cp.start()
    tgt_cp.wait()
    pltpu.sync_copy(rows_v.at[pl.ds(0, b_per)],
                    tgt_hbm.at[pl.ds(base_b, b_per)])

    lec_cp = pltpu.make_async_copy(lecture_hbm.at[doc_v], lec_v, sem)
    lec_cp.start()
    lec_cp.wait()

    b_sub = b_per // 4
    for sub in range(4):
      ctx_cp = [
          pltpu.make_async_copy(
              wemb_hbm.at[ctx_iv.at[pl.ds(sub * b_sub * CTX + j * 128, 128)]],
              rows_v.at[pl.ds(j * 128, 128)], sem)
          for j in range((b_sub * CTX) // 128)
      ]
      for cp in ctx_cp:
        cp.start()
      for cp in ctx_cp:
        cp.wait()

      def acc_loop(i, _, sub=sub):
        b_local = sub * b_sub + i
        for q in range(D // L):
          acc = lec_v[b_local, pl.ds(q * L, L)]
          for j in range(CTX):
            acc = acc + rows_v[i * CTX + j, pl.ds(q * L, L)]
          dvec_v[b_local, pl.ds(q * L, L)] = acc * (1.0 / 21.0)
        return 0
      lax.fori_loop(0, b_sub, acc_loop, 0)

    pltpu.sync_copy(dvec_v, dvec_hbm.at[pl.ds(base_b, b_per)])

  return sc_call


def kernel(inputs, target, lecture, word_emb, freq_dic):
  inputs = inputs.astype(jnp.int32)
  target = target.astype(jnp.int32)

  cdf = jnp.cumsum(freq_dic)
  total = cdf[-1]
  u = jax.random.uniform(jax.random.key(42), (B * NS,), dtype=jnp.float32) * total
  cdf2d = jnp.concatenate(
      [cdf, jnp.full((PAD - VOCAB,), total, jnp.float32)]).reshape(NB, BW)

  info = plsc.get_sparse_core_info()
  dvec, tgt, nvec = _build_sc_call(info.num_cores, info.num_subcores)(
      inputs.reshape(-1), target, lecture, word_emb, cdf2d, u)

  return (dvec.reshape(B, 1, D), tgt.reshape(B, 1, D), nvec.reshape(B, D, NS))


# speedup vs baseline: 1.0465x; 1.0465x over previous
import functools

import jax
import jax.numpy as jnp
from jax import lax
from jax.experimental import pallas as pl
from jax.experimental.pallas import tpu as pltpu
from jax.experimental.pallas import tpu_sc as plsc

B = 4096
CTX = 20
D = 64
VOCAB = 1_000_000
LEC = 100_000
NS = 5

NB = 16_384
BW = 64
PAD = NB * BW
L = 16

_COMPILER_PARAMS = pltpu.CompilerParams(
    needs_layout_passes=False, use_tc_tiling_on_sc=False)


def _mesh(num_cores, num_subcores):
  return plsc.VectorSubcoreMesh(
      core_axis_name="c", subcore_axis_name="s",
      num_cores=num_cores, num_subcores=num_subcores)


def _build_pool_call(num_cores: int, num_subcores: int):
  nw = num_cores * num_subcores
  b_per = B // nw
  b_sub = b_per // 4
  rows_c = b_sub * CTX
  w = CTX + 1

  @functools.partial(
      pl.kernel,
      out_type=[
          jax.ShapeDtypeStruct((B, D), jnp.float32),
          jax.ShapeDtypeStruct((B, D), jnp.float32),
      ],
      mesh=_mesh(num_cores, num_subcores),
      compiler_params=_COMPILER_PARAMS,
      scratch_types=[
          pltpu.VMEM((b_per * w,), jnp.int32),
          pltpu.VMEM((rows_c, D), jnp.float32),
          pltpu.VMEM((rows_c, D), jnp.float32),
          pltpu.VMEM((b_per, D), jnp.float32),
          pltpu.VMEM((b_per, D), jnp.float32),
          pltpu.VMEM((b_per, D), jnp.float32),
          pltpu.VMEM((b_per,), jnp.int32),
          pltpu.VMEM((b_per,), jnp.int32),
          pltpu.VMEM((b_per * CTX,), jnp.int32),
          pltpu.SemaphoreType.DMA,
          pltpu.SemaphoreType.DMA,
          pltpu.SemaphoreType.DMA,
      ],
  )
  def pool_call(inputs_hbm, target_hbm, lecture_hbm, wemb_hbm,
                dvec_hbm, tgt_hbm,
                inputs_v, rows_a, rows_b, tgt_v, lec_v, dvec_v,
                doc_v, tgt_iv, ctx_iv, sem_a, sem_b, sem_m):
    c = lax.axis_index("c")
    s = lax.axis_index("s")
    wid = s * num_cores + c
    base_b = wid * b_per

    pltpu.sync_copy(inputs_hbm.at[pl.ds(base_b * w, b_per * w)], inputs_v)
    pltpu.sync_copy(target_hbm.at[pl.ds(base_b, b_per)], tgt_iv)

    def doc_loop(g, _):
      rows = lax.iota(jnp.int32, L) + g * L
      doc_v[pl.ds(g * L, L)] = plsc.load_gather(inputs_v, [rows * w])
      return 0
    lax.fori_loop(0, b_per // L, doc_loop, 0)

    def ctx_loop(g, _):
      p = lax.iota(jnp.int32, L) + g * L
      r = p // CTX
      cc = p - r * CTX + 1
      ctx_iv[pl.ds(g * L, L)] = plsc.load_gather(inputs_v, [r * w + cc])
      return 0
    lax.fori_loop(0, (b_per * CTX) // L, ctx_loop, 0)

    tgt_cp = pltpu.make_async_copy(wemb_hbm.at[tgt_iv], tgt_v, sem_m)
    lec_cp = pltpu.make_async_copy(lecture_hbm.at[doc_v], lec_v, sem_m)
    tgt_cp.start()
    lec_cp.start()

    bufs = (rows_a, rows_b)
    sems = (sem_a, sem_b)

    def fire(sub):
      buf, sem = bufs[sub % 2], sems[sub % 2]
      cps = [
          pltpu.make_async_copy(
              wemb_hbm.at[ctx_iv.at[pl.ds(sub * rows_c + j * 128, 128)]],
              buf.at[pl.ds(j * 128, 128)], sem)
          for j in range(rows_c // 128)
      ]
      for cp in cps:
        cp.start()
      return cps

    inflight = fire(0)
    lec_cp.wait()
    tgt_cp.wait()

    for sub in range(4):
      for cp in inflight:
        cp.wait()
      cur = bufs[sub % 2]
      if sub < 3:
        inflight = fire(sub + 1)

      def acc_loop(i, _, cur=cur, sub=sub):
        b_local = sub * b_sub + i
        for q in range(D // L):
          acc = lec_v[b_local, pl.ds(q * L, L)]
          for j in range(CTX):
            acc = acc + cur[i * CTX + j, pl.ds(q * L, L)]
          dvec_v[b_local, pl.ds(q * L, L)] = acc * (1.0 / 21.0)
        return 0
      lax.fori_loop(0, b_sub, acc_loop, 0)

    pltpu.sync_copy(dvec_v, dvec_hbm.at[pl.ds(base_b, b_per)])
    pltpu.sync_copy(tgt_v, tgt_hbm.at[pl.ds(base_b, b_per)])

  return pool_call


def _build_sample_call(num_cores: int, num_subcores: int):
  nw = num_cores * num_subcores
  b_per = B // nw
  samp = b_per * NS
  cb = NB // num_subcores
  st = cb // 2

  @functools.partial(
      pl.kernel,
      out_type=jax.ShapeDtypeStruct((B * NS, D), jnp.float32),
      mesh=_mesh(num_cores, num_subcores),
      compiler_params=_COMPILER_PARAMS,
      scratch_types=[
          pltpu.VMEM((st, BW), jnp.float32),
          pltpu.VMEM((st, BW), jnp.float32),
          pltpu.VMEM((cb,), jnp.float32),
          pltpu.VMEM((NB,), jnp.float32),
          pltpu.VMEM_SHARED((NB,), jnp.float32),
          pltpu.VMEM((samp, D), jnp.float32),
          pltpu.VMEM((samp,), jnp.float32),
          pltpu.VMEM((samp,), jnp.int32),
          pltpu.VMEM((samp,), jnp.int32),
          pltpu.SemaphoreType.DMA,
          pltpu.SemaphoreType.DMA,
      ],
  )
  def sample_call(cdf2d_hbm, u_hbm, wemb_hbm, nvec_hbm,
                  stage_a, stage_b, cc_v, coarse_v, shared_coarse,
                  rows_v, u_v, bucket_v, nw_v, sem_a, sem_b):
    c = lax.axis_index("c")
    s = lax.axis_index("s")
    wid = s * num_cores + c

    cp_a = pltpu.make_async_copy(
        cdf2d_hbm.at[pl.ds(s * cb, st)], stage_a, sem_a)
    cp_b = pltpu.make_async_copy(
        cdf2d_hbm.at[pl.ds(s * cb + st, st)], stage_b, sem_b)
    cp_a.start()
    cp_b.start()
    pltpu.sync_copy(u_hbm.at[pl.ds(wid * samp, samp)], u_v)

    for h, (cp, stage) in enumerate(((cp_a, stage_a), (cp_b, stage_b))):
      cp.wait()

      def ext_loop(g, _, h=h, stage=stage):
        rows = lax.iota(jnp.int32, L) + g * L
        cols = jnp.full((L,), BW - 1, jnp.int32)
        cc_v[pl.ds(h * st + g * L, L)] = plsc.load_gather(stage, [rows, cols])
        return 0
      lax.fori_loop(0, st // L, ext_loop, 0)

    pltpu.sync_copy(cc_v, shared_coarse.at[pl.ds(s * cb, cb)])
    plsc.subcore_barrier()
    pltpu.sync_copy(shared_coarse, coarse_v)

    def coarse_search(g, _):
      u = u_v[pl.ds(g * L, L)]
      lo = jnp.zeros((L,), jnp.int32)
      hi = jnp.full((L,), NB - 1, jnp.int32)
      for _ in range(14):
        mid = (lo + hi) >> 1
        vals = plsc.load_gather(coarse_v, [mid])
        pred = vals < u
        lo = jnp.where(pred, mid + 1, lo)
        hi = jnp.where(pred, hi, mid)
      bucket_v[pl.ds(g * L, L)] = lo
      return 0
    lax.fori_loop(0, samp // L, coarse_search, 0)

    win = [
        pltpu.make_async_copy(
            cdf2d_hbm.at[bucket_v.at[pl.ds(j * 128, 128)]],
            rows_v.at[pl.ds(j * 128, 128)], sem_a)
        for j in range(samp // 128)
    ]
    for cp in win:
      cp.start()
    for cp in win:
      cp.wait()

    def fine_search(g, _):
      u = u_v[pl.ds(g * L, L)]
      rows = lax.iota(jnp.int32, L) + g * L
      lo = jnp.zeros((L,), jnp.int32)
      hi = jnp.full((L,), BW - 1, jnp.int32)
      for _ in range(6):
        mid = (lo + hi) >> 1
        vals = plsc.load_gather(rows_v, [rows, mid])
        pred = vals < u
        lo = jnp.where(pred, mid + 1, lo)
        hi = jnp.where(pred, hi, mid)
      bucket = bucket_v[pl.ds(g * L, L)]
      word = jnp.minimum(bucket * BW + lo,
                         jnp.full((L,), VOCAB - 1, jnp.int32))
      nw_v[pl.ds(g * L, L)] = word
      return 0
    lax.fori_loop(0, samp // L, fine_search, 0)

    neg = [
        pltpu.make_async_copy(
            wemb_hbm.at[nw_v.at[pl.ds(j * 128, 128)]],
            rows_v.at[pl.ds(j * 128, 128)], sem_a)
        for j in range(samp // 128)
    ]
    for cp in neg:
      cp.start()
    for cp in neg:
      cp.wait()

    def neg_loop(r, _):
      for q in range(D // L):
        rows_v[r, pl.ds(q * L, L)] = -rows_v[r, pl.ds(q * L, L)]
      return 0
    lax.fori_loop(0, samp, neg_loop, 0)
    pltpu.sync_copy(rows_v, nvec_hbm.at[pl.ds(wid * samp, samp)])

  return sample_call


def kernel(inputs, target, lecture, word_emb, freq_dic):
  inputs = inputs.astype(jnp.int32)
  target = target.astype(jnp.int32)

  cdf = jnp.cumsum(freq_dic)
  total = cdf[-1]
  u = jax.random.uniform(jax.random.key(42), (B * NS,), dtype=jnp.float32) * total
  cdf2d = jnp.concatenate(
      [cdf, jnp.full((PAD - VOCAB,), total, jnp.float32)]).reshape(NB, BW)

  info = plsc.get_sparse_core_info()
  nc, ns = info.num_cores, info.num_subcores
  dvec, tgt = _build_pool_call(nc, ns)(
      inputs.reshape(-1), target, lecture, word_emb)
  nvec = _build_sample_call(nc, ns)(cdf2d, u, word_emb)

  return (dvec.reshape(B, 1, D), tgt.reshape(B, 1, D), nvec.reshape(B, D, NS))
